# baseline (speedup 1.0000x reference)
import functools

import jax
import jax.numpy as jnp
import numpy as np
from jax import lax
from jax.experimental import pallas as pl
from jax.experimental.pallas import tpu as pltpu
from jax.experimental.pallas import tpu_sc as plsc

N_NODES = 10000
DIM = 128
DIMW = DIM // 2
NC = 2
NS = 16
NW = NC * NS
CHUNK = 128

_mesh = plsc.VectorSubcoreMesh(core_axis_name="c", subcore_axis_name="s")

_PERM = np.empty(DIM, np.int32)
for _q in range(DIM // 32):
    for _k in range(16):
        _PERM[32 * _q + 2 * _k] = 32 * _q + _k
        _PERM[32 * _q + 2 * _k + 1] = 32 * _q + 16 + _k


def _make_sc_aggregate(e_pad: int):
    per_w = e_pad // NW
    n_chunks = per_w // CHUNK
    assert n_chunks % 2 == 0
    nc0 = max(2, (n_chunks * 2 * 228 // (228 + 272)) // 2 * 2)
    nc1 = 2 * n_chunks - nc0

    @functools.partial(
        pl.kernel,
        out_type=jax.ShapeDtypeStruct((NC, N_NODES, DIM), jnp.float32),
        mesh=_mesh,
        compiler_params=pltpu.CompilerParams(use_tc_tiling_on_sc=False),
        scratch_types=[
            pltpu.VMEM((2, CHUNK), jnp.int32),
            pltpu.VMEM((2, CHUNK), jnp.int32),
            pltpu.VMEM((2, CHUNK), jnp.float32),
            pltpu.VMEM((CHUNK, DIMW), jnp.int32),
            pltpu.VMEM((CHUNK, DIMW), jnp.int32),
            pltpu.VMEM((CHUNK, DIM), jnp.float32),
            pltpu.VMEM((CHUNK, DIM), jnp.float32),
            pltpu.VMEM_SHARED((N_NODES, DIM), jnp.float32),
            pltpu.SemaphoreType.DMA,
            pltpu.SemaphoreType.DMA,
            pltpu.SemaphoreType.DMA,
            pltpu.SemaphoreType.DMA,
            pltpu.SemaphoreType.DMA,
            pltpu.SemaphoreType.DMA,
        ],
    )
    def _sc_aggregate(x_hbm, src_hbm, dst_hbm, attr_hbm, zeros_hbm, part_hbm,
                      sidx, didx, attrb, rowsi0, rowsi1, rowsf0, rowsf1, acc,
                      si0, si1, sg0, sg1, ss0, ss1):
        cid = lax.axis_index("c")
        sid = lax.axis_index("s")
        w = cid * NS + sid
        rowsi = (rowsi0, rowsi1)
        rowsf = (rowsf0, rowsf1)
        si = (si0, si1)
        sg = (sg0, sg1)
        ss = (ss0, ss1)
        my_n = jnp.where(cid == 0, nc0, nc1)
        row0 = jnp.where(cid == 0, sid * nc0, NS * nc0 + sid * nc1)

        @pl.when(sid < NS - 1)
        def _zero_main():
            pltpu.sync_copy(zeros_hbm.at[pl.ds(sid * 624, 624)],
                            acc.at[pl.ds(sid * 624, 624)])

        @pl.when(sid == NS - 1)
        def _zero_last():
            pltpu.sync_copy(zeros_hbm.at[pl.ds((NS - 1) * 624, 640)],
                            acc.at[pl.ds((NS - 1) * 624, 640)])

        plsc.subcore_barrier()

        def fire_idx(ci, s):
            pltpu.async_copy(src_hbm.at[row0 + ci], sidx.at[s], si[s])
            pltpu.async_copy(dst_hbm.at[row0 + ci], didx.at[s], si[s])
            pltpu.async_copy(attr_hbm.at[row0 + ci], attrb.at[s], si[s])

        def wait_idx(ci, s):
            pltpu.make_async_copy(src_hbm.at[row0 + ci], sidx.at[s],
                                  si[s]).wait()
            pltpu.make_async_copy(dst_hbm.at[row0 + ci], didx.at[s],
                                  si[s]).wait()
            pltpu.make_async_copy(attr_hbm.at[row0 + ci], attrb.at[s],
                                  si[s]).wait()

        def fire_gather(b):
            pltpu.async_copy(x_hbm.at[sidx.at[b]], rowsi[b], sg[b])

        def wait_gather(b):
            pltpu.make_async_copy(x_hbm.at[sidx.at[b]], rowsi[b],
                                  sg[b]).wait()

        def fire_scatter(b):
            pltpu.async_copy(rowsf[b], acc.at[didx.at[b]], ss[b], add=True)

        def wait_scatter(b):
            pltpu.make_async_copy(rowsf[b], acc.at[didx.at[b]], ss[b]).wait()

        fire_idx(0, 0)
        wait_idx(0, 0)
        fire_gather(0)

        mask = jnp.int32(-65536)

        def pair_body(i, carry):
            for b in range(2):
                ci = 2 * i + b
                ob = 1 - b

                @pl.when(ci >= 1)
                def _drain_scatter():
                    wait_scatter(ob)

                @pl.when(ci + 1 < my_n)
                def _prefetch_idx():
                    fire_idx(ci + 1, ob)

                wait_gather(b)

                @pl.when(ci + 1 < my_n)
                def _next_gather():
                    wait_idx(ci + 1, ob)
                    fire_gather(ob)

                @plsc.parallel_loop(0, CHUNK // 16, unroll=2)
                def group_body(g):
                    a16 = attrb[b, pl.ds(g * 16, 16)]
                    for l in range(16):
                        av = jnp.full((16,), a16[l], dtype=jnp.float32)
                        e = g * 16 + l
                        for j in range(DIM // 32):
                            vi = rowsi[b][e, pl.ds(j * 16, 16)]
                            lo = lax.bitcast_convert_type(vi << 16, jnp.float32)
                            hi = lax.bitcast_convert_type(vi & mask, jnp.float32)
                            rowsf[b][e, pl.ds(32 * j, 16)] = lo * av
                            rowsf[b][e, pl.ds(32 * j + 16, 16)] = hi * av

                fire_scatter(b)
            return carry

        lax.fori_loop(0, my_n // 2, pair_body, 0)
        wait_scatter(1)

        plsc.subcore_barrier()
        r0 = sid * 624

        @pl.when(sid < NS - 1)
        def _copy_main():
            pltpu.sync_copy(acc.at[pl.ds(r0, 624)],
                            part_hbm.at[cid, pl.ds(r0, 624)])

        @pl.when(sid == NS - 1)
        def _copy_last():
            pltpu.sync_copy(acc.at[pl.ds((NS - 1) * 624, 640)],
                            part_hbm.at[cid, pl.ds((NS - 1) * 624, 640)])

    return _sc_aggregate


BLK = 1000


def _tc_body(part_ref, w_ref, b_ref, o_ref):
    p = part_ref[0] + part_ref[1]
    y = lax.dot_general(p, w_ref[...], (((1,), (1,)), ((), ())),
                        preferred_element_type=jnp.float32)
    y = y + b_ref[...]
    o_ref[...] = jnp.where(y >= 0.0, y, 0.01 * y)


_tc_linear = pl.pallas_call(
    _tc_body,
    grid=(N_NODES // BLK,),
    in_specs=[
        pl.BlockSpec((NC, BLK, DIM), lambda i: (0, i, 0)),
        pl.BlockSpec((DIM, DIM), lambda i: (0, 0)),
        pl.BlockSpec((1, DIM), lambda i: (0, 0)),
    ],
    out_specs=pl.BlockSpec((BLK, DIM), lambda i: (i, 0)),
    out_shape=jax.ShapeDtypeStruct((N_NODES, DIM), jnp.float32),
)


def kernel(x, edge_index, edge_attr, W, b):
    src = edge_index[0].astype(jnp.int32)
    dst = edge_index[1].astype(jnp.int32)
    attr = edge_attr.astype(jnp.float32)
    n_e = src.shape[0]
    quantum = NW * CHUNK * 2
    e_pad = -(-n_e // quantum) * quantum
    pad = e_pad - n_e
    if pad:
        src = jnp.pad(src, (0, pad))
        dst = jnp.pad(dst, (0, pad))
        attr = jnp.pad(attr, (0, pad))
    n_chunks_total = e_pad // CHUNK
    src = src.reshape(n_chunks_total, CHUNK)
    dst = dst.reshape(n_chunks_total, CHUNK)
    attr = attr.reshape(n_chunks_total, CHUNK)
    xp = x.astype(jnp.bfloat16)[:, jnp.asarray(_PERM)]
    xi = lax.bitcast_convert_type(xp.reshape(x.shape[0], DIMW, 2), jnp.int32)
    zeros = jnp.zeros((N_NODES, DIM), jnp.float32)
    part = _make_sc_aggregate(e_pad)(xi, src, dst, attr, zeros)
    return _tc_linear(part, W, b.reshape(1, DIM))


# speedup vs baseline: 1.1030x; 1.1030x over previous
import functools

import jax
import jax.numpy as jnp
import numpy as np
from jax import lax
from jax.experimental import pallas as pl
from jax.experimental.pallas import tpu as pltpu
from jax.experimental.pallas import tpu_sc as plsc

N_NODES = 10000
DIM = 128
DIMW = DIM // 2
NC = 2
NS = 16
NW = NC * NS
CHUNK = 128

_mesh = plsc.VectorSubcoreMesh(core_axis_name="c", subcore_axis_name="s")

_PERM = np.empty(DIM, np.int32)
for _q in range(DIM // 32):
    for _k in range(16):
        _PERM[32 * _q + 2 * _k] = 32 * _q + _k
        _PERM[32 * _q + 2 * _k + 1] = 32 * _q + 16 + _k


def _make_sc_aggregate(e_pad: int):
    per_w = e_pad // NW
    n_chunks = per_w // CHUNK
    assert n_chunks % 2 == 0
    nc0 = max(2, (n_chunks * 2 * 272 // (228 + 272)) // 2 * 2)
    nc1 = 2 * n_chunks - nc0

    @functools.partial(
        pl.kernel,
        out_type=jax.ShapeDtypeStruct((NC, N_NODES, DIM), jnp.float32),
        mesh=_mesh,
        compiler_params=pltpu.CompilerParams(use_tc_tiling_on_sc=False),
        scratch_types=[
            pltpu.VMEM((2, CHUNK), jnp.int32),
            pltpu.VMEM((2, CHUNK), jnp.int32),
            pltpu.VMEM((2, CHUNK), jnp.float32),
            pltpu.VMEM((CHUNK, DIMW), jnp.int32),
            pltpu.VMEM((CHUNK, DIMW), jnp.int32),
            pltpu.VMEM((CHUNK, DIM), jnp.float32),
            pltpu.VMEM((CHUNK, DIM), jnp.float32),
            pltpu.VMEM_SHARED((N_NODES, DIM), jnp.float32),
            pltpu.SemaphoreType.DMA,
            pltpu.SemaphoreType.DMA,
            pltpu.SemaphoreType.DMA,
            pltpu.SemaphoreType.DMA,
            pltpu.SemaphoreType.DMA,
            pltpu.SemaphoreType.DMA,
        ],
    )
    def _sc_aggregate(x_hbm, src_hbm, dst_hbm, attr_hbm, zeros_hbm, part_hbm,
                      sidx, didx, attrb, rowsi0, rowsi1, rowsf0, rowsf1, acc,
                      si0, si1, sg0, sg1, ss0, ss1):
        cid = lax.axis_index("c")
        sid = lax.axis_index("s")
        w = cid * NS + sid
        rowsi = (rowsi0, rowsi1)
        rowsf = (rowsf0, rowsf1)
        si = (si0, si1)
        sg = (sg0, sg1)
        ss = (ss0, ss1)
        my_n = jnp.where(cid == 0, nc0, nc1)
        row0 = jnp.where(cid == 0, sid * nc0, NS * nc0 + sid * nc1)

        @pl.when(sid < NS - 1)
        def _zero_main():
            pltpu.sync_copy(zeros_hbm.at[pl.ds(sid * 624, 624)],
                            acc.at[pl.ds(sid * 624, 624)])

        @pl.when(sid == NS - 1)
        def _zero_last():
            pltpu.sync_copy(zeros_hbm.at[pl.ds((NS - 1) * 624, 640)],
                            acc.at[pl.ds((NS - 1) * 624, 640)])

        plsc.subcore_barrier()

        def fire_idx(ci, s):
            pltpu.async_copy(src_hbm.at[row0 + ci], sidx.at[s], si[s])
            pltpu.async_copy(dst_hbm.at[row0 + ci], didx.at[s], si[s])
            pltpu.async_copy(attr_hbm.at[row0 + ci], attrb.at[s], si[s])

        def wait_idx(ci, s):
            pltpu.make_async_copy(src_hbm.at[row0 + ci], sidx.at[s],
                                  si[s]).wait()
            pltpu.make_async_copy(dst_hbm.at[row0 + ci], didx.at[s],
                                  si[s]).wait()
            pltpu.make_async_copy(attr_hbm.at[row0 + ci], attrb.at[s],
                                  si[s]).wait()

        def fire_gather(b):
            pltpu.async_copy(x_hbm.at[sidx.at[b]], rowsi[b], sg[b])

        def wait_gather(b):
            pltpu.make_async_copy(x_hbm.at[sidx.at[b]], rowsi[b],
                                  sg[b]).wait()

        def fire_scatter(b):
            pltpu.async_copy(rowsf[b], acc.at[didx.at[b]], ss[b], add=True)

        def wait_scatter(b):
            pltpu.make_async_copy(rowsf[b], acc.at[didx.at[b]], ss[b]).wait()

        fire_idx(0, 0)
        wait_idx(0, 0)
        fire_gather(0)

        mask = jnp.int32(-65536)

        def pair_body(i, carry):
            for b in range(2):
                ci = 2 * i + b
                ob = 1 - b

                @pl.when(ci >= 1)
                def _drain_scatter():
                    wait_scatter(ob)

                @pl.when(ci + 1 < my_n)
                def _prefetch_idx():
                    fire_idx(ci + 1, ob)

                wait_gather(b)

                @pl.when(ci + 1 < my_n)
                def _next_gather():
                    wait_idx(ci + 1, ob)
                    fire_gather(ob)

                @plsc.parallel_loop(0, CHUNK // 16, unroll=2)
                def group_body(g):
                    a16 = attrb[b, pl.ds(g * 16, 16)]
                    for l in range(16):
                        av = jnp.full((16,), a16[l], dtype=jnp.float32)
                        e = g * 16 + l
                        for j in range(DIM // 32):
                            vi = rowsi[b][e, pl.ds(j * 16, 16)]
                            lo = lax.bitcast_convert_type(vi << 16, jnp.float32)
                            hi = lax.bitcast_convert_type(vi & mask, jnp.float32)
                            rowsf[b][e, pl.ds(32 * j, 16)] = lo * av
                            rowsf[b][e, pl.ds(32 * j + 16, 16)] = hi * av

                fire_scatter(b)
            return carry

        lax.fori_loop(0, my_n // 2, pair_body, 0)
        wait_scatter(1)

        plsc.subcore_barrier()
        r0 = sid * 624

        @pl.when(sid < NS - 1)
        def _copy_main():
            pltpu.sync_copy(acc.at[pl.ds(r0, 624)],
                            part_hbm.at[cid, pl.ds(r0, 624)])

        @pl.when(sid == NS - 1)
        def _copy_last():
            pltpu.sync_copy(acc.at[pl.ds((NS - 1) * 624, 640)],
                            part_hbm.at[cid, pl.ds((NS - 1) * 624, 640)])

    return _sc_aggregate


BLK = 1000


def _tc_body(part_ref, w_ref, b_ref, o_ref):
    p = part_ref[0] + part_ref[1]
    y = lax.dot_general(p, w_ref[...], (((1,), (1,)), ((), ())),
                        preferred_element_type=jnp.float32)
    y = y + b_ref[...]
    o_ref[...] = jnp.where(y >= 0.0, y, 0.01 * y)


_tc_linear = pl.pallas_call(
    _tc_body,
    grid=(N_NODES // BLK,),
    in_specs=[
        pl.BlockSpec((NC, BLK, DIM), lambda i: (0, i, 0)),
        pl.BlockSpec((DIM, DIM), lambda i: (0, 0)),
        pl.BlockSpec((1, DIM), lambda i: (0, 0)),
    ],
    out_specs=pl.BlockSpec((BLK, DIM), lambda i: (i, 0)),
    out_shape=jax.ShapeDtypeStruct((N_NODES, DIM), jnp.float32),
)


def kernel(x, edge_index, edge_attr, W, b):
    src = edge_index[0].astype(jnp.int32)
    dst = edge_index[1].astype(jnp.int32)
    attr = edge_attr.astype(jnp.float32)
    n_e = src.shape[0]
    quantum = NW * CHUNK * 2
    e_pad = -(-n_e // quantum) * quantum
    pad = e_pad - n_e
    if pad:
        src = jnp.pad(src, (0, pad))
        dst = jnp.pad(dst, (0, pad))
        attr = jnp.pad(attr, (0, pad))
    n_chunks_total = e_pad // CHUNK
    src = src.reshape(n_chunks_total, CHUNK)
    dst = dst.reshape(n_chunks_total, CHUNK)
    attr = attr.reshape(n_chunks_total, CHUNK)
    xp = x.astype(jnp.bfloat16)[:, jnp.asarray(_PERM)]
    xi = lax.bitcast_convert_type(xp.reshape(x.shape[0], DIMW, 2), jnp.int32)
    zeros = jnp.zeros((N_NODES, DIM), jnp.float32)
    part = _make_sc_aggregate(e_pad)(xi, src, dst, attr, zeros)
    return _tc_linear(part, W, b.reshape(1, DIM))
